# Initial kernel scaffold; baseline (speedup 1.0000x reference)
import jax
import jax.numpy as jnp
from jax.experimental import pallas as pl


def kernel(inputs):
    raise NotImplementedError("write your pallas kernel here")


# speedup vs baseline: 11.2114x; 11.2114x over previous
import jax
import jax.numpy as jnp
import numpy as np
from jax.experimental import pallas as pl

_K = 64
_ROWS = 128
_COLS = 32768
_BLOCK_ROWS = 8


def _ksparse_block(x_ref, o_ref):
    x = x_ref[...]
    s = jax.lax.bitcast_convert_type(x, jnp.int32)
    min32 = jnp.int32(-2147483648)
    ikey = jnp.where(s < 0, jnp.bitwise_xor(jnp.invert(s), min32), s)
    prefix = jnp.full((x.shape[0], 1), -2147483648, jnp.int32)
    for b in range(31, -1, -1):
        inc = jnp.int32(np.uint32(1 << b).astype(np.int32))
        cand = prefix + inc
        cnt = jnp.sum((ikey >= cand).astype(jnp.int32), axis=1, keepdims=True)
        prefix = jnp.where(cnt >= _K, cand, prefix)
    o_ref[...] = jnp.where(ikey >= prefix, x, jnp.float32(0.0))


def kernel(inputs):
    grid = (_ROWS // _BLOCK_ROWS,)
    return pl.pallas_call(
        _ksparse_block,
        grid=grid,
        in_specs=[pl.BlockSpec((_BLOCK_ROWS, _COLS), lambda i: (i, 0))],
        out_specs=pl.BlockSpec((_BLOCK_ROWS, _COLS), lambda i: (i, 0)),
        out_shape=jax.ShapeDtypeStruct((_ROWS, _COLS), jnp.float32),
    )(inputs)
